# baseline (speedup 1.0000x reference)
import functools
import jax
import jax.numpy as jnp
from jax import lax
from jax.experimental import pallas as pl
from jax.experimental.pallas import tpu as pltpu
from jax.experimental.pallas import tpu_sc as plsc

NW = 32
L = 16
CAPB = 256
GCHUNK = 64
N_GROUPS = 100000
N_MOD = 3200000
NMB = N_MOD // 16
WLIMB = NMB - CAPB
GPW = N_GROUPS // NW
NB = (GPW + L - 1) // L
GPAD = ((NB * L + GCHUNK - 1) // GCHUNK) * GCHUNK
NCH = GPAD // GCHUNK
CSR_LEN = NB * L + 8
D_MOD = 16
BIGD = 128
CW = GCHUNK * D_MOD


def _sc_body(csr_hbm, xmap_hbm, xmod_hbm, out_rows,
             csr_v, win_v, args_v, sub_v, seen_v, bg0, bg1, rows_v,
             sg0, sg1):
    c = lax.axis_index("c")
    s = lax.axis_index("s")
    wid = s * 2 + c
    g0 = wid * GPW
    a0 = pl.multiple_of((g0 // 8) * 8, 8)
    off = g0 - a0
    pltpu.sync_copy(csr_hbm.at[pl.ds(a0, CSR_LEN)], csr_v)

    iot = lax.iota(jnp.int32, L)
    neg = jnp.float32(jnp.finfo(jnp.float32).min)
    negv = jnp.full((L,), neg, jnp.float32)
    sent = jnp.full((L,), N_MOD, jnp.int32)
    zl = jnp.zeros((L,), jnp.int32)

    def bcast(x):
        return jnp.broadcast_to(x, (L,))

    stv0 = csr_v[pl.ds(off, L)]
    w_init = pl.multiple_of(
        jnp.minimum(((stv0[0] >> 4) // 8) * 8, WLIMB), 8)
    pltpu.sync_copy(xmap_hbm.at[pl.ds(w_init, CAPB)], win_v)

    def batch_body(b, w_cur):
        bb = b * L
        stv = csr_v[pl.ds(off + bb, L)]
        env = csr_v[pl.ds(off + bb + 1, L)]
        env = jnp.where((bcast(bb) + iot) < bcast(GPW), env, stv)
        seenv = env > stv
        bstart = stv[0]
        bend = jnp.max(env)
        kmax = jnp.max(env - stv)

        def store_results(argv):
            argv = jnp.where(seenv, argv, zl)
            args_v[pl.ds(bb, L)] = argv >> 3
            sub_v[pl.ds(bb, L)] = (argv & 7) * D_MOD
            seen_v[pl.ds(bb, L)] = jnp.where(
                seenv, jnp.full((L,), 1.0, jnp.float32),
                jnp.zeros((L,), jnp.float32))

        def good(w):
            def reload(a):
                wn = pl.multiple_of(
                    jnp.minimum(((a >> 4) // 8) * 8, WLIMB), 8)
                pltpu.sync_copy(xmap_hbm.at[pl.ds(wn, CAPB)], win_v)
                return wn

            w = lax.cond((bend + 15) >> 4 > w + CAPB,
                         lambda a: reload(a[0]), lambda a: a[1],
                         (bstart, w))

            def stepk(k, carry):
                am, ai = carry
                idx = stv + bcast(k)
                m = idx < env
                relr = jnp.where(m, (idx >> 4) - bcast(w), zl)
                colv = jnp.where(m, (idx & 15) * 8, zl)
                v = plsc.load_gather(win_v, [relr, colv])
                v = jnp.where(m, v, negv)
                upd = v > am
                am = jnp.where(upd, v, am)
                ai = jnp.where(upd, idx, ai)
                return (am, ai)

            _, ai = lax.fori_loop(0, kmax, stepk, (negv, sent))
            store_results(ai)
            return w

        def bad(w):
            argv = zl
            for j in range(L):
                st = stv[j]
                en = env[j]

                def cond_fn(carry):
                    return carry[0] < en

                def step_fn(carry):
                    p, w2, am, ai = carry

                    def reload(a):
                        wn = pl.multiple_of(
                            jnp.minimum(((a >> 4) // 8) * 8, WLIMB),
                            8)
                        pltpu.sync_copy(
                            xmap_hbm.at[pl.ds(wn, CAPB)], win_v)
                        return wn

                    w2 = lax.cond((p + L + 15) >> 4 > w2 + CAPB,
                                  lambda a: reload(a[0]),
                                  lambda a: a[1], (p, w2))
                    idx = bcast(p) + iot
                    m = idx < bcast(en)
                    relr = jnp.where(m, (idx >> 4) - bcast(w2), zl)
                    colv = jnp.where(m, (idx & 15) * 8, zl)
                    v = plsc.load_gather(win_v, [relr, colv])
                    v = jnp.where(m, v, negv)
                    upd = v > am
                    am = jnp.where(upd, v, am)
                    ai = jnp.where(upd, idx, ai)
                    return (p + L, w2, am, ai)

                _, w, am, ai = lax.while_loop(
                    cond_fn, step_fn, (st, w, negv, sent))
                gmax = jnp.max(am)
                cand = jnp.where(am == bcast(gmax), ai, sent)
                arg = jnp.min(cand)
                argv = jnp.where(iot == j, bcast(arg), argv)
            store_results(argv)
            return w

        return lax.cond(((bend + 15) >> 4) - (bstart >> 4) <= CAPB - 8, good, bad, w_cur)

    lax.fori_loop(0, NB, batch_body, w_init)

    def pad_body(b, _):
        args_v[pl.ds(b * L, L)] = zl
        sub_v[pl.ds(b * L, L)] = zl
        seen_v[pl.ds(b * L, L)] = jnp.zeros((L,), jnp.float32)
        return 0

    lax.fori_loop(NB, GPAD // L, pad_body, 0)

    def gather_src(cix):
        base = pl.multiple_of(cix * GCHUNK, 8)
        return xmod_hbm.at[args_v.at[pl.ds(base, GCHUNK)]]

    def extract(cix, bg):
        cb = cix * GCHUNK
        for b2 in range(GCHUNK // L):
            sov = sub_v[pl.ds(cb + b2 * L, L)]
            sfv = seen_v[pl.ds(cb + b2 * L, L)]
            for jj in range(L):
                r = b2 * L + jj
                row = bg[r, pl.ds(sov[jj], D_MOD)]
                rows_v[pl.ds((cb + r) * D_MOD, D_MOD)] = (
                    row * bcast(sfv[jj]))

    def phase2_body(i, _):
        pltpu.async_copy(gather_src(i), bg0, sg0).wait()
        extract(i, bg0)
        return 0

    lax.fori_loop(0, NCH, phase2_body, 0)

    obase = pl.multiple_of(wid * (GPW * D_MOD), 8)
    pltpu.sync_copy(rows_v.at[pl.ds(0, GPW * D_MOD)],
                    out_rows.at[pl.ds(obase, GPW * D_MOD)])


@functools.partial(
    pl.kernel,
    mesh=plsc.VectorSubcoreMesh(core_axis_name="c", subcore_axis_name="s"),
    compiler_params=pltpu.CompilerParams(needs_layout_passes=False),
    out_type=jax.ShapeDtypeStruct((NW * GPW * D_MOD,), jnp.float32),
    scratch_types=[
        pltpu.VMEM((CSR_LEN,), jnp.int32),
        pltpu.VMEM((CAPB, BIGD), jnp.float32),
        pltpu.VMEM((GPAD,), jnp.int32),
        pltpu.VMEM((GPAD,), jnp.int32),
        pltpu.VMEM((GPAD,), jnp.float32),
        pltpu.VMEM((GCHUNK, BIGD), jnp.float32),
        pltpu.VMEM((GCHUNK, BIGD), jnp.float32),
        pltpu.VMEM((GPAD * D_MOD,), jnp.float32),
        pltpu.SemaphoreType.DMA,
        pltpu.SemaphoreType.DMA,
    ],
)
def _sc_pool(csr_hbm, xmap_hbm, xmod_hbm, out_rows,
             csr_v, win_v, args_v, sub_v, seen_v, bg0, bg1, rows_v,
             sg0, sg1):
    _sc_body(csr_hbm, xmap_hbm, xmod_hbm, out_rows,
             csr_v, win_v, args_v, sub_v, seen_v, bg0, bg1, rows_v,
             sg0, sg1)


@jax.jit
def kernel(x_main, x_mod, x_map, csr_idx):
    del x_main
    csr = csr_idx.astype(jnp.int32)
    csr_pad = jnp.concatenate([csr, jnp.zeros((32,), jnp.int32)])
    xmap_big = x_map.reshape(-1, BIGD)
    xmod_big = x_mod.reshape(-1, BIGD)
    out_rows = _sc_pool(csr_pad, xmap_big, xmod_big)
    x_pool = out_rows.reshape(N_GROUPS, D_MOD)
    x_seen = csr_idx[1:] > csr_idx[:-1]
    return (x_pool, x_seen)


# speedup vs baseline: 6.5084x; 6.5084x over previous
import functools
import jax
import jax.numpy as jnp
from jax import lax
from jax.experimental import pallas as pl
from jax.experimental.pallas import tpu as pltpu
from jax.experimental.pallas import tpu_sc as plsc

NW = 32
L = 16
CAPW = 4096
CAPC = 2048
N_GROUPS = 100000
N_MOD = 3200000
WLIMW = N_MOD - CAPW
WLIMC = N_MOD - CAPC
GPW = N_GROUPS // NW
NB = (GPW + L - 1) // L
GPAD = NB * L
CSR_LEN = GPAD + 8
D_MOD = 16


def _sc_body(csr_hbm, xmapt_hbm, xmodt_hbm, out_rows,
             csr_v, win_v, args_v, seen_v, win2_v, rows_v, sem2):
    c = lax.axis_index("c")
    s = lax.axis_index("s")
    wid = s * 2 + c
    g0 = wid * GPW
    a0 = pl.multiple_of((g0 // 8) * 8, 8)
    off = g0 - a0
    pltpu.sync_copy(csr_hbm.at[pl.ds(a0, CSR_LEN)], csr_v)

    iot = lax.iota(jnp.int32, L)
    neg = jnp.float32(jnp.finfo(jnp.float32).min)
    negv = jnp.full((L,), neg, jnp.float32)
    sent = jnp.full((L,), N_MOD, jnp.int32)
    zl = jnp.zeros((L,), jnp.int32)

    def bcast(x):
        return jnp.broadcast_to(x, (L,))

    stv0 = csr_v[pl.ds(off, L)]
    start0 = stv0[0]
    w_init = pl.multiple_of(
        jnp.minimum((start0 // 128) * 128, WLIMW), 128)
    pltpu.sync_copy(
        xmapt_hbm.at[pl.ds(0, 8), pl.ds(w_init, CAPW)], win_v)

    def batch_body(b, w_cur):
        bb = b * L
        stv = csr_v[pl.ds(off + bb, L)]
        env = csr_v[pl.ds(off + bb + 1, L)]
        env = jnp.where((bcast(bb) + iot) < bcast(GPW), env, stv)
        seenv = env > stv
        bstart = stv[0]
        bend = jnp.max(env)
        kmax = jnp.max(env - stv)

        def store_results(argv):
            argv = jnp.where(seenv, argv, zl)
            args_v[pl.ds(bb, L)] = argv
            seen_v[pl.ds(bb, L)] = jnp.where(
                seenv, jnp.full((L,), 1.0, jnp.float32),
                jnp.zeros((L,), jnp.float32))

        def reload(a):
            wn = pl.multiple_of(
                jnp.minimum((a // 128) * 128, WLIMW), 128)
            pltpu.sync_copy(
                xmapt_hbm.at[pl.ds(0, 8), pl.ds(wn, CAPW)], win_v)
            return wn

        def good(w):
            w = lax.cond(bend > w + CAPW,
                         lambda a: reload(a[0]), lambda a: a[1],
                         (bstart, w))

            def stepk(k, carry):
                am, ai = carry
                idx = stv + bcast(k)
                m = idx < env
                rel = jnp.where(m, idx - bcast(w), zl)
                v = plsc.load_gather(win_v, [zl, rel])
                v = jnp.where(m, v, negv)
                upd = v > am
                am = jnp.where(upd, v, am)
                ai = jnp.where(upd, idx, ai)
                return (am, ai)

            _, ai = lax.fori_loop(0, kmax, stepk, (negv, sent))
            store_results(ai)
            return w

        def bad(w):
            argv = zl
            for j in range(L):
                st = stv[j]
                en = env[j]

                def cond_fn(carry):
                    return carry[0] < en

                def step_fn(carry):
                    p, w2, am, ai = carry
                    w2 = lax.cond(p + L > w2 + CAPW,
                                  lambda a: reload(a[0]),
                                  lambda a: a[1], (p, w2))
                    idx = bcast(p) + iot
                    m = idx < bcast(en)
                    rel = jnp.where(m, idx - bcast(w2), zl)
                    v = plsc.load_gather(win_v, [zl, rel])
                    v = jnp.where(m, v, negv)
                    upd = v > am
                    am = jnp.where(upd, v, am)
                    ai = jnp.where(upd, idx, ai)
                    return (p + L, w2, am, ai)

                _, w, am, ai = lax.while_loop(
                    cond_fn, step_fn, (st, w, negv, sent))
                gmax = jnp.max(am)
                cand = jnp.where(am == bcast(gmax), ai, sent)
                arg = jnp.min(cand)
                argv = jnp.where(iot == j, bcast(arg), argv)
            store_results(argv)
            return w

        return lax.cond(bend - bstart <= CAPW - 128, good, bad, w_cur)

    lax.fori_loop(0, NB, batch_body, w_init)

    def reload2(a):
        wn = pl.multiple_of(
            jnp.minimum((a // 128) * 128, WLIMC), 128)
        pltpu.sync_copy(
            xmodt_hbm.at[pl.ds(0, D_MOD), pl.ds(wn, CAPC)], win2_v)
        return wn

    wc_init = reload2(start0)

    def p2_body(b, wc):
        bb = b * L
        argv = args_v[pl.ds(bb, L)]
        sfv = seen_v[pl.ds(bb, L)]
        for j in range(L):
            a = argv[j]
            wc = lax.cond(a >= wc + CAPC,
                          lambda t: reload2(t[0]), lambda t: t[1],
                          (a, wc))
            relc = jnp.maximum(a - wc, 0)
            row = plsc.load_gather(win2_v, [iot, bcast(relc)])
            rows_v[pl.ds((bb + j) * D_MOD, D_MOD)] = (
                row * bcast(sfv[j]))
        return wc

    lax.fori_loop(0, NB, p2_body, wc_init)

    obase = pl.multiple_of(wid * (GPW * D_MOD), 8)
    pltpu.sync_copy(rows_v.at[pl.ds(0, GPW * D_MOD)],
                    out_rows.at[pl.ds(obase, GPW * D_MOD)])


@functools.partial(
    pl.kernel,
    mesh=plsc.VectorSubcoreMesh(core_axis_name="c", subcore_axis_name="s"),
    compiler_params=pltpu.CompilerParams(needs_layout_passes=False),
    out_type=jax.ShapeDtypeStruct((NW * GPW * D_MOD,), jnp.float32),
    scratch_types=[
        pltpu.VMEM((CSR_LEN,), jnp.int32),
        pltpu.VMEM((8, CAPW), jnp.float32),
        pltpu.VMEM((GPAD,), jnp.int32),
        pltpu.VMEM((GPAD,), jnp.float32),
        pltpu.VMEM((D_MOD, CAPC), jnp.float32),
        pltpu.VMEM((GPAD * D_MOD,), jnp.float32),
        pltpu.SemaphoreType.DMA,
    ],
)
def _sc_pool(csr_hbm, xmapt_hbm, xmodt_hbm, out_rows,
             csr_v, win_v, args_v, seen_v, win2_v, rows_v, sem2):
    _sc_body(csr_hbm, xmapt_hbm, xmodt_hbm, out_rows,
             csr_v, win_v, args_v, seen_v, win2_v, rows_v, sem2)


@jax.jit
def kernel(x_main, x_mod, x_map, csr_idx):
    del x_main
    csr = csr_idx.astype(jnp.int32)
    csr_pad = jnp.concatenate([csr, jnp.zeros((32,), jnp.int32)])
    out_rows = _sc_pool(csr_pad, x_map.T, x_mod.T)
    x_pool = out_rows.reshape(N_GROUPS, D_MOD)
    x_seen = csr_idx[1:] > csr_idx[:-1]
    return (x_pool, x_seen)
